# baseline (speedup 1.0000x reference)
import jax
import jax.numpy as jnp
from jax import lax
from jax.experimental import pallas as pl
from jax.experimental.pallas import tpu as pltpu
from jax.experimental.pallas import tpu_sc as plsc

B = 4096 * 200
D = 64
NW = 32
BPW = B // NW
C = 128
NCHUNK = BPW // C
NB = 4
LA = 2
SCALE = 8.0


def _body(idx_hbm, table_hbm, out_hbm, idx_v, idx2_v, wide_v, *sems):
    gsems = sems[0:NB]
    ssems = sems[NB:2 * NB]
    wid = lax.axis_index("s") * 2 + lax.axis_index("c")
    base = wid * BPW

    pltpu.sync_copy(idx_hbm.at[pl.ds(base, BPW)], idx_v)

    def issue_gather(g, b):
        for k in range(C // 16):
            sl = pl.ds(k * 16, 16)
            idx2_v[b, sl] = lax.shift_right_logical(
                idx_v[pl.ds(g * C + k * 16, 16)], 1)
        pltpu.async_copy(table_hbm.at[idx2_v.at[b]], wide_v.at[b], gsems[b])

    def wait_gather(b):
        pltpu.make_async_copy(table_hbm.at[idx2_v.at[b]], wide_v.at[b],
                              gsems[b]).wait()

    def issue_scatter(g, b):
        off = pl.multiple_of((base + g * C) // 2, C // 2)
        pltpu.async_copy(wide_v.at[b, pl.ds(0, C // 2)],
                         out_hbm.at[pl.ds(off, C // 2)],
                         ssems[b])

    def wait_scatter(b):
        off = pl.multiple_of(base // 2, C // 2)
        pltpu.make_async_copy(wide_v.at[b, pl.ds(0, C // 2)],
                              out_hbm.at[pl.ds(off, C // 2)],
                              ssems[b]).wait()

    for g in range(LA):
        issue_gather(g, g % NB)

    @pl.loop(0, NCHUNK, step=NB)
    def _(t):
        for b in range(NB):
            g = t + b
            wait_gather(b)

            bb = (b + LA) % NB

            @pl.when(g + LA < NCHUNK)
            def _():
                @pl.when(g >= NB - LA)
                def _():
                    wait_scatter(bb)
                issue_gather(g + LA, bb)

            def group(m, carry):
                par = (idx_v[pl.ds(g * C + m * 16, 16)] & 1) * D
                for t in range(8):
                    p0 = par[2 * t]
                    p1 = par[2 * t + 1]
                    k = m * 8 + t
                    for j in range(D // 16):
                        wide_v[b, k, pl.ds(j * 16, 16)] = (
                            wide_v[b, 2 * k, pl.ds(p0 + j * 16, 16)] * SCALE)
                    for j in range(D // 16):
                        wide_v[b, k, pl.ds(D + j * 16, 16)] = (
                            wide_v[b, 2 * k + 1, pl.ds(p1 + j * 16, 16)]
                            * SCALE)
                return carry

            lax.fori_loop(0, C // 16, group, 0)
            issue_scatter(g, b)

    for b in range(NB):
        wait_scatter(b)


def kernel(x, table):
    xf = x.reshape(-1).astype(jnp.int32)
    table_w = table.reshape(500000, 2 * D)
    out = pl.kernel(
        _body,
        mesh=plsc.VectorSubcoreMesh(core_axis_name="c", subcore_axis_name="s"),
        compiler_params=pltpu.CompilerParams(use_tc_tiling_on_sc=True),
        out_type=jax.ShapeDtypeStruct((B // 2, 2 * D), jnp.float32),
        scratch_types=[
            pltpu.VMEM((BPW,), jnp.int32),
            pltpu.VMEM((NB, C), jnp.int32),
            pltpu.VMEM((NB, C, 2 * D), jnp.float32),
        ] + [pltpu.SemaphoreType.DMA] * (2 * NB),
    )(xf, table_w)
    return out.reshape(x.shape[0], x.shape[1], D)


# speedup vs baseline: 1.2850x; 1.2850x over previous
import jax
import jax.numpy as jnp
from jax import lax
from jax.experimental import pallas as pl
from jax.experimental.pallas import tpu as pltpu
from jax.experimental.pallas import tpu_sc as plsc

B = 4096 * 200
D = 64
NW = 32
BPW = B // NW
C = 128
NCHUNK = BPW // C
NB = 4
NS = 2
LA = 2
SCALE = 8.0


def _body(idx_hbm, table_hbm, out_hbm, idx_v, wide_v, stg_v, *sems):
    isems = sems[0:NB]
    gsems = sems[NB:2 * NB]
    ssems = sems[2 * NB:2 * NB + NS]
    wid = lax.axis_index("s") * 2 + lax.axis_index("c")
    base = wid * BPW

    def issue_idx(g, b):
        off = pl.multiple_of(base + g * C, C)
        pltpu.async_copy(idx_hbm.at[pl.ds(off, C)], idx_v.at[b], isems[b])

    def wait_idx(b):
        off = pl.multiple_of(base, C)
        pltpu.make_async_copy(idx_hbm.at[pl.ds(off, C)], idx_v.at[b],
                              isems[b]).wait()

    def issue_gather(g, b):
        pltpu.async_copy(table_hbm.at[idx_v.at[b]], wide_v.at[b], gsems[b])

    def wait_gather(b):
        pltpu.make_async_copy(table_hbm.at[idx_v.at[b]], wide_v.at[b],
                              gsems[b]).wait()

    def issue_scatter(g, b):
        off = pl.multiple_of((base + g * C) // 2, C // 2)
        pltpu.async_copy(stg_v.at[b], out_hbm.at[pl.ds(off, C // 2)],
                         ssems[b])

    def wait_scatter(b):
        off = pl.multiple_of(base // 2, C // 2)
        pltpu.make_async_copy(stg_v.at[b], out_hbm.at[pl.ds(off, C // 2)],
                              ssems[b]).wait()

    for g in range(LA + 1):
        issue_idx(g, g % NB)
    for g in range(LA):
        wait_idx(g % NB)
        issue_gather(g, g % NB)

    @pl.loop(0, NCHUNK, step=NB)
    def _(t):
        for b in range(NB):
            g = t + b
            wait_gather(b)

            bi = (b + LA + 1) % NB
            bg = (b + LA) % NB
            bs = b % NS

            @pl.when(g + LA + 1 < NCHUNK)
            def _():
                issue_idx(g + LA + 1, bi)

            @pl.when(g + LA < NCHUNK)
            def _():
                wait_idx(bg)
                issue_gather(g + LA, bg)

            @pl.when(g >= NS)
            def _():
                wait_scatter(bs)

            @plsc.parallel_loop(0, C // 2, step=1, unroll=2)
            def _(k):
                for j in range(D // 16):
                    sl = pl.ds(j * 16, 16)
                    stg_v[bs, k, sl] = wide_v[b, 2 * k, sl] * SCALE
                for j in range(D // 16):
                    stg_v[bs, k, pl.ds(D + j * 16, 16)] = (
                        wide_v[b, 2 * k + 1, pl.ds(j * 16, 16)] * SCALE)

            issue_scatter(g, bs)

    for b in range(NS):
        wait_scatter(b)


def kernel(x, table):
    xf = x.reshape(-1).astype(jnp.int32)
    table_p = jnp.pad(table, ((0, 0), (0, D)))
    out = pl.kernel(
        _body,
        mesh=plsc.VectorSubcoreMesh(core_axis_name="c", subcore_axis_name="s"),
        compiler_params=pltpu.CompilerParams(use_tc_tiling_on_sc=True),
        out_type=jax.ShapeDtypeStruct((B // 2, 2 * D), jnp.float32),
        scratch_types=[
            pltpu.VMEM((NB, C), jnp.int32),
            pltpu.VMEM((NB, C, 2 * D), jnp.float32),
            pltpu.VMEM((NS, C // 2, 2 * D), jnp.float32),
        ] + [pltpu.SemaphoreType.DMA] * (2 * NB + NS),
    )(xf, table_p)
    return out.reshape(x.shape[0], x.shape[1], D)
